# baseline (speedup 1.0000x reference)
import functools

import jax
import jax.numpy as jnp
import numpy as np
from jax import lax
from jax.experimental import pallas as pl
from jax.experimental.pallas import tpu as pltpu
from jax.experimental.pallas import tpu_sc as plsc

N_SAMPLES = 1024
LOSS_WEIGHT = 0.0001
NC = 2
NS = 16
L = 16
NW = NC * NS
S_PER_W = N_SAMPLES // NW
ROWS_PER_W = S_PER_W * 4


def _rotl(x, d):
    return ((x << np.uint32(d)) | (x >> np.uint32(32 - d))).astype(np.uint32)


def _threefry2x32(k1, k2, x1, x2):
    rot = [(13, 15, 26, 6), (17, 29, 16, 24)]
    ks = [np.uint32(k1), np.uint32(k2),
          np.uint32(k1) ^ np.uint32(k2) ^ np.uint32(0x1BD11BDA)]
    x = [np.asarray(x1, np.uint32) + ks[0], np.asarray(x2, np.uint32) + ks[1]]
    order = [(rot[0], ks[1], ks[2], 1), (rot[1], ks[2], ks[0], 2),
             (rot[0], ks[0], ks[1], 3), (rot[1], ks[1], ks[2], 4),
             (rot[0], ks[2], ks[0], 5)]
    for rots, ka, kb, c in order:
        for r in rots:
            x[0] = (x[0] + x[1]).astype(np.uint32)
            x[1] = x[0] ^ _rotl(x[1], r)
        x[0] = (x[0] + ka).astype(np.uint32)
        x[1] = (x[1] + kb + np.uint32(c)).astype(np.uint32)
    return x[0], x[1]


def _iota_2x32(n):
    i = np.arange(n, dtype=np.uint64)
    return ((i >> np.uint64(32)).astype(np.uint32),
            (i & np.uint64(0xFFFFFFFF)).astype(np.uint32))


def _split2(k1, k2):
    hi, lo = _iota_2x32(2)
    b1, b2 = _threefry2x32(k1, k2, hi, lo)
    return (b1[0], b2[0]), (b1[1], b2[1])


@functools.lru_cache(maxsize=None)
def _flat_indices(k: int) -> np.ndarray:
    root = (np.uint32(0), np.uint32(42))
    ku, kv = _split2(*root)
    span = np.uint32(k)
    mult = np.uint32((pow(2, 16, k) * pow(2, 16, k)) % k)
    parts = []
    for kk in (ku, kv):
        k1, k2 = _split2(*kk)
        hi, lo = _iota_2x32(N_SAMPLES * 4)
        hb = np.bitwise_xor(*_threefry2x32(k1[0], k1[1], hi, lo))
        lb = np.bitwise_xor(*_threefry2x32(k2[0], k2[1], hi, lo))
        off = ((hb % span) * mult + lb % span) % span
        parts.append(off.astype(np.int32).reshape(NW, ROWS_PER_W))
    return np.concatenate(parts, axis=1)


NCH = 8
CROWS = ROWS_PER_W // NCH
CSAMP = S_PER_W // NCH


@functools.lru_cache(maxsize=None)
def _make_gather_kernel(r: int):
    ch = r // L

    mesh = plsc.VectorSubcoreMesh(
        core_axis_name="c", subcore_axis_name="s", num_cores=NC, num_subcores=NS
    )

    @functools.partial(
        pl.kernel,
        out_type=jax.ShapeDtypeStruct((NW, L), jnp.float32),
        mesh=mesh,
        scratch_types=[
            pltpu.VMEM((2 * ROWS_PER_W,), jnp.int32),
            pltpu.VMEM((2, CROWS, r), jnp.float32),
            pltpu.VMEM((L,), jnp.float32),
            pltpu.SemaphoreType.DMA,
            pltpu.SemaphoreType.DMA,
        ],
    )
    def gather_kernel(u_hbm, v_hbm, idx_hbm, out_hbm,
                      idx_v, rows_v, stats_v, sem0, sem1):
        wid = lax.axis_index("s") * NC + lax.axis_index("c")
        lanes = lax.iota(jnp.int32, L)

        dnums = lax.GatherDimensionNumbers(
            offset_dims=(), collapsed_slice_dims=(0,), start_index_map=(0,)
        )

        def lane_total(x):
            for s in (1, 2, 4, 8):
                perm = lax.gather(
                    x, (lanes ^ s)[:, None], dnums, (1,),
                    mode=lax.GatherScatterMode.PROMISE_IN_BOUNDS,
                )
                x = x + perm
            return x

        pltpu.sync_copy(idx_hbm.at[wid], idx_v)

        tables = [u_hbm, v_hbm]
        sems = [sem0, sem1]

        def start(cb, k, buf):
            isl = idx_v.at[pl.ds(cb * ROWS_PER_W + k * CROWS, CROWS)]
            pltpu.async_copy(tables[cb].at[isl], rows_v.at[buf], sems[buf])

        def wait(cb, buf):
            pltpu.make_async_copy(
                tables[cb].at[idx_v.at[pl.ds(cb * ROWS_PER_W, CROWS)]],
                rows_v.at[buf], sems[buf],
            ).wait()

        def chunk_stats(b, carry):
            def sample_body(j, cin):
                s1, s2, sa = cin

                def chunk_body(cc, acc):
                    a = rows_v[b, 4 * j + 0, pl.ds(cc * L, L)]
                    bb = rows_v[b, 4 * j + 1, pl.ds(cc * L, L)]
                    cx = rows_v[b, 4 * j + 2, pl.ds(cc * L, L)]
                    d = rows_v[b, 4 * j + 3, pl.ds(cc * L, L)]
                    return acc + (a * bb) * (cx * d)

                acc = lax.fori_loop(
                    0, ch, chunk_body, jnp.zeros((L,), jnp.float32), unroll=2
                )
                sim = lane_total(acc) * jnp.float32(1.0 / r)
                return (s1 + sim, s2 + sim * sim, sa + jnp.abs(sim))

            return lax.fori_loop(0, CSAMP, sample_body, carry)

        z3 = (jnp.zeros((L,), jnp.float32),) * 3
        stats_cb = []
        start(0, 0, 0)
        start(0, 1, 1)
        for cb in (0, 1):

            def pair_body(p, stats, cb=cb):
                wait(cb, 0)
                stats = chunk_stats(0, stats)

                @pl.when(2 * p + 2 < NCH)
                def _():
                    start(cb, 2 * p + 2, 0)

                if cb == 0:
                    @pl.when(2 * p + 2 >= NCH)
                    def _():
                        start(1, 0, 0)

                wait(cb, 1)
                stats = chunk_stats(1, stats)

                @pl.when(2 * p + 3 < NCH)
                def _():
                    start(cb, 2 * p + 3, 1)

                if cb == 0:
                    @pl.when(2 * p + 3 >= NCH)
                    def _():
                        start(1, 1, 1)

                return stats

            stats_cb.append(lax.fori_loop(0, NCH // 2, pair_body, z3))

        s1u, s2u, sau = stats_cb[0]
        s1v, s2v, sav = stats_cb[1]

        zeros = jnp.zeros((L,), jnp.float32)
        stats = (
            jnp.where(lanes == 0, s1u, zeros)
            + jnp.where(lanes == 1, s2u, zeros)
            + jnp.where(lanes == 2, sau, zeros)
            + jnp.where(lanes == 3, s1v, zeros)
            + jnp.where(lanes == 4, s2v, zeros)
            + jnp.where(lanes == 5, sav, zeros)
        )
        stats_v[...] = stats
        pltpu.sync_copy(stats_v, out_hbm.at[wid])

    return gather_kernel


def _combine_body(p_ref, o_ref):
    tot = jnp.sum(p_ref[...], axis=0)
    lanes = lax.iota(jnp.int32, L)

    def pick(i):
        return jnp.sum(jnp.where(lanes == i, tot, 0.0))

    n = jnp.float32(N_SAMPLES)
    s1u, s2u, sau = pick(0), pick(1), pick(2)
    s1v, s2v, sav = pick(3), pick(4), pick(5)
    var_u = (s2u - s1u * s1u / n) / (n - 1.0)
    var_v = (s2v - s1v * s1v / n) / (n - 1.0)
    loss_u = -(sau / n) + 0.1 * (1.0 - var_u)
    loss_v = -(sav / n) + 0.1 * (1.0 - var_v)
    loss = (loss_u + loss_v) * LOSS_WEIGHT
    o_ref[...] = jnp.full((1, 1), loss, dtype=jnp.float32)


def kernel(codebook_U, codebook_V):
    k, r = codebook_U.shape
    idx = jnp.asarray(_flat_indices(k))
    partials = _make_gather_kernel(r)(codebook_U, codebook_V, idx)
    out = pl.pallas_call(
        _combine_body,
        out_shape=jax.ShapeDtypeStruct((1, 1), jnp.float32),
    )(partials)
    return out[0, 0]


# speedup vs baseline: 1.0495x; 1.0495x over previous
import functools

import jax
import jax.numpy as jnp
import numpy as np
from jax import lax
from jax.experimental import pallas as pl
from jax.experimental.pallas import tpu as pltpu
from jax.experimental.pallas import tpu_sc as plsc

N_SAMPLES = 1024
LOSS_WEIGHT = 0.0001
NC = 2
NS = 16
L = 16
NW = NC * NS
S_PER_W = N_SAMPLES // NW
ROWS_PER_W = S_PER_W * 4


def _rotl(x, d):
    return ((x << np.uint32(d)) | (x >> np.uint32(32 - d))).astype(np.uint32)


def _threefry2x32(k1, k2, x1, x2):
    rot = [(13, 15, 26, 6), (17, 29, 16, 24)]
    ks = [np.uint32(k1), np.uint32(k2),
          np.uint32(k1) ^ np.uint32(k2) ^ np.uint32(0x1BD11BDA)]
    x = [np.asarray(x1, np.uint32) + ks[0], np.asarray(x2, np.uint32) + ks[1]]
    order = [(rot[0], ks[1], ks[2], 1), (rot[1], ks[2], ks[0], 2),
             (rot[0], ks[0], ks[1], 3), (rot[1], ks[1], ks[2], 4),
             (rot[0], ks[2], ks[0], 5)]
    for rots, ka, kb, c in order:
        for r in rots:
            x[0] = (x[0] + x[1]).astype(np.uint32)
            x[1] = x[0] ^ _rotl(x[1], r)
        x[0] = (x[0] + ka).astype(np.uint32)
        x[1] = (x[1] + kb + np.uint32(c)).astype(np.uint32)
    return x[0], x[1]


def _iota_2x32(n):
    i = np.arange(n, dtype=np.uint64)
    return ((i >> np.uint64(32)).astype(np.uint32),
            (i & np.uint64(0xFFFFFFFF)).astype(np.uint32))


def _split2(k1, k2):
    hi, lo = _iota_2x32(2)
    b1, b2 = _threefry2x32(k1, k2, hi, lo)
    return (b1[0], b2[0]), (b1[1], b2[1])


@functools.lru_cache(maxsize=None)
def _flat_indices(k: int) -> np.ndarray:
    root = (np.uint32(0), np.uint32(42))
    ku, kv = _split2(*root)
    span = np.uint32(k)
    mult = np.uint32((pow(2, 16, k) * pow(2, 16, k)) % k)
    parts = []
    for kk in (ku, kv):
        k1, k2 = _split2(*kk)
        hi, lo = _iota_2x32(N_SAMPLES * 4)
        hb = np.bitwise_xor(*_threefry2x32(k1[0], k1[1], hi, lo))
        lb = np.bitwise_xor(*_threefry2x32(k2[0], k2[1], hi, lo))
        off = ((hb % span) * mult + lb % span) % span
        parts.append(off.astype(np.int32).reshape(NW, ROWS_PER_W))
    return np.concatenate(parts, axis=1)


NCH = 4
CROWS = ROWS_PER_W // NCH
CSAMP = S_PER_W // NCH


@functools.lru_cache(maxsize=None)
def _make_gather_kernel(r: int):
    ch = r // L

    mesh = plsc.VectorSubcoreMesh(
        core_axis_name="c", subcore_axis_name="s", num_cores=NC, num_subcores=NS
    )

    @functools.partial(
        pl.kernel,
        out_type=jax.ShapeDtypeStruct((NW, L), jnp.float32),
        mesh=mesh,
        scratch_types=[
            pltpu.VMEM((2 * ROWS_PER_W,), jnp.int32),
            pltpu.VMEM((2, CROWS, r), jnp.float32),
            pltpu.VMEM((L,), jnp.float32),
            pltpu.SemaphoreType.DMA,
            pltpu.SemaphoreType.DMA,
        ],
    )
    def gather_kernel(u_hbm, v_hbm, idx_hbm, out_hbm,
                      idx_v, rows_v, stats_v, sem0, sem1):
        wid = lax.axis_index("s") * NC + lax.axis_index("c")
        lanes = lax.iota(jnp.int32, L)

        dnums = lax.GatherDimensionNumbers(
            offset_dims=(), collapsed_slice_dims=(0,), start_index_map=(0,)
        )

        def lane_total(x):
            for s in (1, 2, 4, 8):
                perm = lax.gather(
                    x, (lanes ^ s)[:, None], dnums, (1,),
                    mode=lax.GatherScatterMode.PROMISE_IN_BOUNDS,
                )
                x = x + perm
            return x

        pltpu.sync_copy(idx_hbm.at[wid], idx_v)

        tables = [u_hbm, v_hbm]
        sems = [sem0, sem1]

        def start(cb, k, buf):
            isl = idx_v.at[pl.ds(cb * ROWS_PER_W + k * CROWS, CROWS)]
            pltpu.async_copy(tables[cb].at[isl], rows_v.at[buf], sems[buf])

        def wait(cb, buf):
            pltpu.make_async_copy(
                tables[cb].at[idx_v.at[pl.ds(cb * ROWS_PER_W, CROWS)]],
                rows_v.at[buf], sems[buf],
            ).wait()

        def chunk_stats(b, carry):
            def sample_body(j, cin):
                s1, s2, sa = cin

                def chunk_body(cc, acc):
                    a = rows_v[b, 4 * j + 0, pl.ds(cc * L, L)]
                    bb = rows_v[b, 4 * j + 1, pl.ds(cc * L, L)]
                    cx = rows_v[b, 4 * j + 2, pl.ds(cc * L, L)]
                    d = rows_v[b, 4 * j + 3, pl.ds(cc * L, L)]
                    return acc + (a * bb) * (cx * d)

                acc = lax.fori_loop(
                    0, ch, chunk_body, jnp.zeros((L,), jnp.float32), unroll=2
                )
                sim = lane_total(acc) * jnp.float32(1.0 / r)
                return (s1 + sim, s2 + sim * sim, sa + jnp.abs(sim))

            return lax.fori_loop(0, CSAMP, sample_body, carry)

        z3 = (jnp.zeros((L,), jnp.float32),) * 3
        stats_cb = []
        start(0, 0, 0)
        start(0, 1, 1)
        for cb in (0, 1):

            def pair_body(p, stats, cb=cb):
                wait(cb, 0)
                stats = chunk_stats(0, stats)

                @pl.when(2 * p + 2 < NCH)
                def _():
                    start(cb, 2 * p + 2, 0)

                if cb == 0:
                    @pl.when(2 * p + 2 >= NCH)
                    def _():
                        start(1, 0, 0)

                wait(cb, 1)
                stats = chunk_stats(1, stats)

                @pl.when(2 * p + 3 < NCH)
                def _():
                    start(cb, 2 * p + 3, 1)

                if cb == 0:
                    @pl.when(2 * p + 3 >= NCH)
                    def _():
                        start(1, 1, 1)

                return stats

            stats_cb.append(lax.fori_loop(0, NCH // 2, pair_body, z3))

        s1u, s2u, sau = stats_cb[0]
        s1v, s2v, sav = stats_cb[1]

        zeros = jnp.zeros((L,), jnp.float32)
        stats = (
            jnp.where(lanes == 0, s1u, zeros)
            + jnp.where(lanes == 1, s2u, zeros)
            + jnp.where(lanes == 2, sau, zeros)
            + jnp.where(lanes == 3, s1v, zeros)
            + jnp.where(lanes == 4, s2v, zeros)
            + jnp.where(lanes == 5, sav, zeros)
        )
        stats_v[...] = stats
        pltpu.sync_copy(stats_v, out_hbm.at[wid])

    return gather_kernel


def _combine_body(p_ref, o_ref):
    tot = jnp.sum(p_ref[...], axis=0)
    lanes = lax.iota(jnp.int32, L)

    def pick(i):
        return jnp.sum(jnp.where(lanes == i, tot, 0.0))

    n = jnp.float32(N_SAMPLES)
    s1u, s2u, sau = pick(0), pick(1), pick(2)
    s1v, s2v, sav = pick(3), pick(4), pick(5)
    var_u = (s2u - s1u * s1u / n) / (n - 1.0)
    var_v = (s2v - s1v * s1v / n) / (n - 1.0)
    loss_u = -(sau / n) + 0.1 * (1.0 - var_u)
    loss_v = -(sav / n) + 0.1 * (1.0 - var_v)
    loss = (loss_u + loss_v) * LOSS_WEIGHT
    o_ref[...] = jnp.full((1, 1), loss, dtype=jnp.float32)


def kernel(codebook_U, codebook_V):
    k, r = codebook_U.shape
    idx = jnp.asarray(_flat_indices(k))
    partials = _make_gather_kernel(r)(codebook_U, codebook_V, idx)
    out = pl.pallas_call(
        _combine_body,
        out_shape=jax.ShapeDtypeStruct((1, 1), jnp.float32),
    )(partials)
    return out[0, 0]
